# baseline (speedup 1.0000x reference)
import jax
import jax.numpy as jnp
from jax.experimental import pallas as pl
from jax.experimental.pallas import tpu as pltpu

_DN = (((1,), (1,)), ((), ()))


def _dot(a, b):
    return jax.lax.dot_general(a, b, _DN, preferred_element_type=jnp.float32)


def _body(xo_ref, xd_ref, wo_ref, wd_ref, o_ref, d_ref, o_acc, d_acc):
    b = pl.program_id(0)
    nb = pl.num_programs(0)
    N = o_ref.shape[0]
    nh = N // 2

    for acc, a_ref, w_ref in ((o_acc, xo_ref, wo_ref),
                              (d_acc, xd_ref, wd_ref)):
        af = a_ref[0].astype(jnp.float32)
        a = af.astype(jnp.bfloat16)
        aw = (af * w_ref[...]).astype(jnp.bfloat16)
        tt = _dot(aw[:nh], a[:nh])
        tb = _dot(aw[:nh], a[nh:])
        bb = _dot(aw[nh:], a[nh:])

        @pl.when(b == 0)
        def _init(acc=acc, tt=tt, tb=tb, bb=bb):
            acc[:nh, :nh] = tt
            acc[:nh, nh:] = tb
            acc[nh:, nh:] = bb

        @pl.when(b > 0)
        def _accum(acc=acc, tt=tt, tb=tb, bb=bb):
            acc[:nh, :nh] += tt
            acc[:nh, nh:] += tb
            acc[nh:, nh:] += bb

    @pl.when(b == nb - 1)
    def _finish():
        for acc, out in ((o_acc, o_ref), (d_acc, d_ref)):
            acc[nh:, :nh] = jnp.transpose(acc[:nh, nh:])
            z = jnp.maximum(acc[...], 0.0)
            z = z - jnp.max(z, axis=1, keepdims=True)
            e = jnp.exp(z)
            out[...] = e / jnp.sum(e, axis=1, keepdims=True)


def kernel(x_t, W_o, W_d):
    B, N, _, H = x_t.shape
    K = N * H
    xo = x_t.reshape(B, N, K)
    xd = (x_t.astype(jnp.bfloat16)
          .swapaxes(1, 2).reshape(B, N, K))
    wo = jnp.repeat(jnp.diagonal(W_o), H).reshape(1, K)
    wd = jnp.repeat(jnp.diagonal(W_d), H).reshape(1, K)

    out_shape = (jax.ShapeDtypeStruct((N, N), jnp.float32),
                 jax.ShapeDtypeStruct((N, N), jnp.float32))
    blk = pl.BlockSpec((1, N, K), lambda b: (b, 0, 0))
    wblk = pl.BlockSpec((1, K), lambda b: (0, 0))
    o, d = pl.pallas_call(
        _body,
        grid=(B,),
        in_specs=[blk, blk, wblk, wblk],
        out_specs=[
            pl.BlockSpec((N, N), lambda b: (0, 0)),
            pl.BlockSpec((N, N), lambda b: (0, 0)),
        ],
        out_shape=out_shape,
        scratch_shapes=[
            pltpu.VMEM((N, N), jnp.float32),
            pltpu.VMEM((N, N), jnp.float32),
        ],
    )(xo, xd, wo, wd)
    return (o, d)


# speedup vs baseline: 1.0710x; 1.0710x over previous
import jax
import jax.numpy as jnp
from jax.experimental import pallas as pl
from jax.experimental.pallas import tpu as pltpu

_DN = (((1,), (1,)), ((), ()))


def _dot(a, b):
    return jax.lax.dot_general(a, b, _DN, preferred_element_type=jnp.float32)


def _body(xo_ref, xd_ref, wo_ref, wd_ref, o_ref, d_ref, o_acc, d_acc):
    b = pl.program_id(0)
    nb = pl.num_programs(0)
    N = o_ref.shape[0]
    nh = N // 2

    for acc, a_ref, w_ref in ((o_acc, xo_ref, wo_ref),
                              (d_acc, xd_ref, wd_ref)):
        a = a_ref[0]
        aw = (a.astype(jnp.float32) * w_ref[...]).astype(jnp.bfloat16)
        tt = _dot(aw[:nh], a[:nh])
        tb = _dot(aw[:nh], a[nh:])
        bb = _dot(aw[nh:], a[nh:])

        @pl.when(b == 0)
        def _init(acc=acc, tt=tt, tb=tb, bb=bb):
            acc[:nh, :nh] = tt
            acc[:nh, nh:] = tb
            acc[nh:, nh:] = bb

        @pl.when(b > 0)
        def _accum(acc=acc, tt=tt, tb=tb, bb=bb):
            acc[:nh, :nh] += tt
            acc[:nh, nh:] += tb
            acc[nh:, nh:] += bb

    @pl.when(b == nb - 1)
    def _finish():
        for acc, out in ((o_acc, o_ref), (d_acc, d_ref)):
            acc[nh:, :nh] = jnp.transpose(acc[:nh, nh:])
            z = jnp.maximum(acc[...], 0.0)
            z = z - jnp.max(z, axis=1, keepdims=True)
            e = jnp.exp(z)
            out[...] = e / jnp.sum(e, axis=1, keepdims=True)


def kernel(x_t, W_o, W_d):
    B, N, _, H = x_t.shape
    K = N * H
    xo32 = jax.lax.optimization_barrier(x_t.reshape(B, N, K))
    xo = xo32.astype(jnp.bfloat16)
    xd = (x_t.astype(jnp.bfloat16)
          .swapaxes(1, 2).reshape(B, N, K))
    wo = jnp.repeat(jnp.diagonal(W_o), H).reshape(1, K)
    wd = jnp.repeat(jnp.diagonal(W_d), H).reshape(1, K)

    out_shape = (jax.ShapeDtypeStruct((N, N), jnp.float32),
                 jax.ShapeDtypeStruct((N, N), jnp.float32))
    blk = pl.BlockSpec((1, N, K), lambda b: (b, 0, 0))
    wblk = pl.BlockSpec((1, K), lambda b: (0, 0))
    o, d = pl.pallas_call(
        _body,
        grid=(B,),
        in_specs=[blk, blk, wblk, wblk],
        out_specs=[
            pl.BlockSpec((N, N), lambda b: (0, 0)),
            pl.BlockSpec((N, N), lambda b: (0, 0)),
        ],
        out_shape=out_shape,
        scratch_shapes=[
            pltpu.VMEM((N, N), jnp.float32),
            pltpu.VMEM((N, N), jnp.float32),
        ],
    )(xo, xd, wo, wd)
    return (o, d)


# speedup vs baseline: 1.1134x; 1.0397x over previous
import jax
import jax.numpy as jnp
from jax.experimental import pallas as pl
from jax.experimental.pallas import tpu as pltpu

_DN = (((1,), (1,)), ((), ()))


def _dot(a, b):
    return jax.lax.dot_general(a, b, _DN, preferred_element_type=jnp.float32)


def _body(xo_ref, xd_ref, wo_ref, wd_ref, o_ref, d_ref, o_acc, d_acc):
    b = pl.program_id(0)
    nb = pl.num_programs(0)
    N = o_ref.shape[0]
    nh = N // 2

    for acc, a_ref, w_ref in ((o_acc, xo_ref, wo_ref),
                              (d_acc, xd_ref, wd_ref)):
        af = a_ref[0]
        a = af.astype(jnp.bfloat16)
        aw = (af * w_ref[...]).astype(jnp.bfloat16)
        tt = _dot(aw[:nh], a[:nh])
        tb = _dot(aw[:nh], a[nh:])
        bb = _dot(aw[nh:], a[nh:])

        @pl.when(b == 0)
        def _init(acc=acc, tt=tt, tb=tb, bb=bb):
            acc[:nh, :nh] = tt
            acc[:nh, nh:] = tb
            acc[nh:, nh:] = bb

        @pl.when(b > 0)
        def _accum(acc=acc, tt=tt, tb=tb, bb=bb):
            acc[:nh, :nh] += tt
            acc[:nh, nh:] += tb
            acc[nh:, nh:] += bb

    @pl.when(b == nb - 1)
    def _finish():
        for acc, out in ((o_acc, o_ref), (d_acc, d_ref)):
            acc[nh:, :nh] = jnp.transpose(acc[:nh, nh:])
            z = jnp.maximum(acc[...], 0.0)
            z = z - jnp.max(z, axis=1, keepdims=True)
            e = jnp.exp(z)
            out[...] = e / jnp.sum(e, axis=1, keepdims=True)


def kernel(x_t, W_o, W_d):
    B, N, _, H = x_t.shape
    K = N * H
    xo = x_t.reshape(B, N, K)
    xd = x_t.swapaxes(1, 2).reshape(B, N, K)
    wo = jnp.repeat(jnp.diagonal(W_o), H).reshape(1, K)
    wd = jnp.repeat(jnp.diagonal(W_d), H).reshape(1, K)

    out_shape = (jax.ShapeDtypeStruct((N, N), jnp.float32),
                 jax.ShapeDtypeStruct((N, N), jnp.float32))
    blk = pl.BlockSpec((1, N, K), lambda b: (b, 0, 0))
    wblk = pl.BlockSpec((1, K), lambda b: (0, 0))
    o, d = pl.pallas_call(
        _body,
        grid=(B,),
        in_specs=[blk, blk, wblk, wblk],
        out_specs=[
            pl.BlockSpec((N, N), lambda b: (0, 0)),
            pl.BlockSpec((N, N), lambda b: (0, 0)),
        ],
        out_shape=out_shape,
        scratch_shapes=[
            pltpu.VMEM((N, N), jnp.float32),
            pltpu.VMEM((N, N), jnp.float32),
        ],
    )(xo, xd, wo, wd)
    return (o, d)
